# baseline (speedup 1.0000x reference)
import functools

import jax
import jax.numpy as jnp
from jax import lax
from jax.experimental import pallas as pl
from jax.experimental.pallas import tpu as pltpu
from jax.experimental.pallas import tpu_sc as plsc

E = 8
D = 1024
F = 4096
T = 2048
C = 384
EC = E * C
PAD_ROWS = (E + 1) * C

NC = 2
NS = 16
NW = NC * NS
CHUNK = T // NW
NG = CHUNK // 16

_MESH = dict(core_axis_name="c", subcore_axis_name="s",
             num_cores=NC, num_subcores=NS)


@functools.partial(
    pl.kernel,
    out_type=(
        jax.ShapeDtypeStruct((PAD_ROWS, D), jnp.float32),
        jax.ShapeDtypeStruct((T,), jnp.int32),
        jax.ShapeDtypeStruct((16,), jnp.int32),
    ),
    mesh=plsc.VectorSubcoreMesh(**_MESH),
    compiler_params=pltpu.CompilerParams(needs_layout_passes=False),
    scratch_types=[
        pltpu.VMEM((T,), jnp.int32),
        pltpu.VMEM((CHUNK, D), jnp.float32),
        pltpu.VMEM((CHUNK,), jnp.int32),
        pltpu.VMEM((16,), jnp.int32),
        pltpu.SemaphoreType.DMA,
        pltpu.SemaphoreType.DMA,
    ],
)
def _route_scatter(eids_hbm, tokens_hbm, xpad_hbm, dest_hbm, nblk_hbm,
                   eids_v, rows_v, dest_v, nblk_v, sem_in, sem_out):
    wid = lax.axis_index("s") * NC + lax.axis_index("c")
    base = wid * CHUNK

    rows_cp = pltpu.make_async_copy(tokens_hbm.at[pl.ds(base, CHUNK)],
                                    rows_v, sem_in)
    rows_cp.start()
    pltpu.sync_copy(eids_hbm, eids_v)

    zero16 = jnp.zeros((16,), jnp.int32)

    def scan_body(g, carry):
        vec = eids_v[pl.ds(g * 16, 16)]
        return tuple(carry[e] + jnp.sum(jnp.where(vec == e, 1, 0))
                     for e in range(E))

    bases = lax.fori_loop(0, wid * NG, scan_body,
                          tuple(jnp.int32(0) for _ in range(E)))

    for g in range(NG):
        vec = eids_v[pl.ds(base + g * 16, 16)]
        pos = zero16
        new_bases = []
        for e in range(E):
            m = vec == e
            onehot = jnp.where(m, 1, 0).astype(jnp.int32)
            csum = plsc.cumsum(onehot)
            pos = jnp.where(m, csum - 1 + bases[e], pos)
            new_bases.append(bases[e] + jnp.sum(onehot))
        bases = tuple(new_bases)
        valid = pos < C
        dest = jnp.where(valid, vec * C + pos, EC + ((pos - C) % C))
        dest_v[pl.ds(g * 16, 16)] = dest

    lane = lax.iota(jnp.int32, 16)
    cnt = zero16
    for e in range(E):
        cnt = jnp.where(lane == e, bases[e], cnt)
    nblk_v[...] = jnp.where(lane < E,
                            (jnp.minimum(cnt, C) + 127) // 128, 0)

    @pl.when(wid == NW - 1)
    def _():
        pltpu.sync_copy(nblk_v, nblk_hbm)

    pltpu.sync_copy(dest_v, dest_hbm.at[pl.ds(base, CHUNK)])
    rows_cp.wait()
    pltpu.async_copy(rows_v, xpad_hbm.at[dest_v], sem_out).wait()


FB = 1024
NF = F // FB
MB = 128
NM = C // MB


def _ffn_body(nblk_ref, x_ref, w1_ref, w3_ref, w2_ref, out_ref):
    e = pl.program_id(0)
    f = pl.program_id(1)

    @pl.when(e < E)
    def _():
        x = x_ref[...].astype(jnp.bfloat16)
        w1b = w1_ref[0].astype(jnp.bfloat16)
        w3b = w3_ref[0].astype(jnp.bfloat16)
        w2b = w2_ref[0].astype(jnp.bfloat16)
        g = jnp.dot(x, w1b, preferred_element_type=jnp.float32)
        u = jnp.dot(x, w3b, preferred_element_type=jnp.float32)
        h = (g * jax.nn.sigmoid(g) * u).astype(jnp.bfloat16)
        contrib = jnp.dot(h, w2b, preferred_element_type=jnp.float32)

        @pl.when(f == 0)
        def _():
            out_ref[...] = contrib

        @pl.when(f > 0)
        def _():
            out_ref[...] = out_ref[...] + contrib

    @pl.when(jnp.logical_and(e == E, f == 0))
    def _():
        out_ref[...] = jnp.zeros_like(out_ref)


def _ffn(x_pad, nblk, w1, w3, w2):
    grid_spec = pltpu.PrefetchScalarGridSpec(
        num_scalar_prefetch=1,
        grid=(E + 1, NF),
        in_specs=[
            pl.BlockSpec((C, D), lambda e, f, nb: (jnp.minimum(e, E - 1), 0)),
            pl.BlockSpec((1, D, FB),
                         lambda e, f, nb: (jnp.minimum(e, E - 1), 0,
                                           jnp.where(e == E, NF - 1, f))),
            pl.BlockSpec((1, D, FB),
                         lambda e, f, nb: (jnp.minimum(e, E - 1), 0,
                                           jnp.where(e == E, NF - 1, f))),
            pl.BlockSpec((1, FB, D),
                         lambda e, f, nb: (jnp.minimum(e, E - 1),
                                           jnp.where(e == E, NF - 1, f), 0)),
        ],
        out_specs=pl.BlockSpec((C, D), lambda e, f, nb: (e, 0)),
    )
    return pl.pallas_call(
        _ffn_body,
        grid_spec=grid_spec,
        out_shape=jax.ShapeDtypeStruct((PAD_ROWS, D), jnp.float32),
        compiler_params=pltpu.CompilerParams(
            vmem_limit_bytes=110 * 1024 * 1024),
    )(nblk, x_pad, w1, w3, w2)


@functools.partial(
    pl.kernel,
    out_type=jax.ShapeDtypeStruct((T, D), jnp.float32),
    mesh=plsc.VectorSubcoreMesh(**_MESH),
    compiler_params=pltpu.CompilerParams(needs_layout_passes=False),
    scratch_types=[
        pltpu.VMEM((CHUNK,), jnp.int32),
        pltpu.VMEM((CHUNK, D), jnp.float32),
        pltpu.SemaphoreType.DMA,
    ],
)
def _unpermute(dest_hbm, outpad_hbm, out_hbm, idx_v, rows_v, sem):
    wid = lax.axis_index("s") * NC + lax.axis_index("c")
    base = wid * CHUNK
    pltpu.sync_copy(dest_hbm.at[pl.ds(base, CHUNK)], idx_v)
    pltpu.async_copy(outpad_hbm.at[idx_v], rows_v, sem).wait()
    pltpu.sync_copy(rows_v, out_hbm.at[pl.ds(base, CHUNK)])


def kernel(tokens, expert_ids, w1, w3, w2):
    eids = expert_ids.astype(jnp.int32)
    x_pad, dest, nblk = _route_scatter(eids, tokens)
    out_pad = _ffn(x_pad, nblk, w1, w3, w2)
    return _unpermute(dest, out_pad)


# speedup vs baseline: 1.0297x; 1.0297x over previous
import functools

import jax
import jax.numpy as jnp
from jax import lax
from jax.experimental import pallas as pl
from jax.experimental.pallas import tpu as pltpu
from jax.experimental.pallas import tpu_sc as plsc

E = 8
D = 1024
F = 4096
T = 2048
C = 384
EC = E * C
PAD_ROWS = (E + 1) * C

NC = 2
NS = 16
NW = NC * NS
CHUNK = T // NW
NG = CHUNK // 16

_MESH = dict(core_axis_name="c", subcore_axis_name="s",
             num_cores=NC, num_subcores=NS)


@functools.partial(
    pl.kernel,
    out_type=(
        jax.ShapeDtypeStruct((PAD_ROWS, D), jnp.float32),
        jax.ShapeDtypeStruct((T,), jnp.int32),
        jax.ShapeDtypeStruct((16,), jnp.int32),
    ),
    mesh=plsc.VectorSubcoreMesh(**_MESH),
    compiler_params=pltpu.CompilerParams(needs_layout_passes=False),
    scratch_types=[
        pltpu.VMEM((T,), jnp.int32),
        pltpu.VMEM((CHUNK, D), jnp.float32),
        pltpu.VMEM((CHUNK,), jnp.int32),
        pltpu.VMEM((16,), jnp.int32),
        pltpu.SemaphoreType.DMA,
        pltpu.SemaphoreType.DMA,
    ],
)
def _route_scatter(eids_hbm, tokens_hbm, xpad_hbm, dest_hbm, nblk_hbm,
                   eids_v, rows_v, dest_v, nblk_v, sem_in, sem_out):
    wid = lax.axis_index("s") * NC + lax.axis_index("c")
    base = wid * CHUNK

    rows_cp = pltpu.make_async_copy(tokens_hbm.at[pl.ds(base, CHUNK)],
                                    rows_v, sem_in)
    rows_cp.start()
    pltpu.sync_copy(eids_hbm, eids_v)

    zero16 = jnp.zeros((16,), jnp.int32)

    def scan_body(g, carry):
        vec = eids_v[pl.ds(g * 16, 16)]
        return tuple(carry[e] + jnp.sum(jnp.where(vec == e, 1, 0))
                     for e in range(E))

    bases = lax.fori_loop(0, wid * NG, scan_body,
                          tuple(jnp.int32(0) for _ in range(E)))

    for g in range(NG):
        vec = eids_v[pl.ds(base + g * 16, 16)]
        pos = zero16
        new_bases = []
        for e in range(E):
            m = vec == e
            onehot = jnp.where(m, 1, 0).astype(jnp.int32)
            csum = plsc.cumsum(onehot)
            pos = jnp.where(m, csum - 1 + bases[e], pos)
            new_bases.append(bases[e] + jnp.sum(onehot))
        bases = tuple(new_bases)
        valid = pos < C
        dest = jnp.where(valid, vec * C + pos, EC + ((pos - C) % C))
        dest_v[pl.ds(g * 16, 16)] = dest

    lane = lax.iota(jnp.int32, 16)
    cnt = zero16
    for e in range(E):
        cnt = jnp.where(lane == e, bases[e], cnt)
    nblk_v[...] = jnp.where(lane < E,
                            (jnp.minimum(cnt, C) + 127) // 128, 0)

    @pl.when(wid == NW - 1)
    def _():
        pltpu.sync_copy(nblk_v, nblk_hbm)

    pltpu.sync_copy(dest_v, dest_hbm.at[pl.ds(base, CHUNK)])
    rows_cp.wait()
    pltpu.async_copy(rows_v, xpad_hbm.at[dest_v], sem_out).wait()


FB = 2048
NF = F // FB
MB = 128
NM = C // MB


def _ffn_body(nblk_ref, x_ref, w1_ref, w3_ref, w2_ref, out_ref):
    e = pl.program_id(0)
    f = pl.program_id(1)

    SB = 1024

    @pl.when(e < E)
    def _():
        x = x_ref[...].astype(jnp.bfloat16)
        contrib = jnp.zeros((C, D), jnp.float32)
        for s in range(FB // SB):
            w1b = w1_ref[0, :, pl.ds(s * SB, SB)].astype(jnp.bfloat16)
            w3b = w3_ref[0, :, pl.ds(s * SB, SB)].astype(jnp.bfloat16)
            w2b = w2_ref[0, pl.ds(s * SB, SB), :].astype(jnp.bfloat16)
            g = jnp.dot(x, w1b, preferred_element_type=jnp.float32)
            u = jnp.dot(x, w3b, preferred_element_type=jnp.float32)
            h = (g * jax.nn.sigmoid(g) * u).astype(jnp.bfloat16)
            contrib = contrib + jnp.dot(h, w2b,
                                        preferred_element_type=jnp.float32)

        @pl.when(f == 0)
        def _():
            out_ref[...] = contrib

        @pl.when(f > 0)
        def _():
            out_ref[...] = out_ref[...] + contrib

    @pl.when(jnp.logical_and(e == E, f == 0))
    def _():
        out_ref[...] = jnp.zeros_like(out_ref)


def _ffn(x_pad, nblk, w1, w3, w2):
    grid_spec = pltpu.PrefetchScalarGridSpec(
        num_scalar_prefetch=1,
        grid=(E + 1, NF),
        in_specs=[
            pl.BlockSpec((C, D), lambda e, f, nb: (jnp.minimum(e, E - 1), 0)),
            pl.BlockSpec((1, D, FB),
                         lambda e, f, nb: (jnp.minimum(e, E - 1), 0,
                                           jnp.where(e == E, NF - 1, f))),
            pl.BlockSpec((1, D, FB),
                         lambda e, f, nb: (jnp.minimum(e, E - 1), 0,
                                           jnp.where(e == E, NF - 1, f))),
            pl.BlockSpec((1, FB, D),
                         lambda e, f, nb: (jnp.minimum(e, E - 1),
                                           jnp.where(e == E, NF - 1, f), 0)),
        ],
        out_specs=pl.BlockSpec((C, D), lambda e, f, nb: (e, 0)),
    )
    return pl.pallas_call(
        _ffn_body,
        grid_spec=grid_spec,
        out_shape=jax.ShapeDtypeStruct((PAD_ROWS, D), jnp.float32),
        compiler_params=pltpu.CompilerParams(
            vmem_limit_bytes=110 * 1024 * 1024),
    )(nblk, x_pad, w1, w3, w2)


@functools.partial(
    pl.kernel,
    out_type=jax.ShapeDtypeStruct((T, D), jnp.float32),
    mesh=plsc.VectorSubcoreMesh(**_MESH),
    compiler_params=pltpu.CompilerParams(needs_layout_passes=False),
    scratch_types=[
        pltpu.VMEM((CHUNK,), jnp.int32),
        pltpu.VMEM((CHUNK, D), jnp.float32),
        pltpu.SemaphoreType.DMA,
    ],
)
def _unpermute(dest_hbm, outpad_hbm, out_hbm, idx_v, rows_v, sem):
    wid = lax.axis_index("s") * NC + lax.axis_index("c")
    base = wid * CHUNK
    pltpu.sync_copy(dest_hbm.at[pl.ds(base, CHUNK)], idx_v)
    pltpu.async_copy(outpad_hbm.at[idx_v], rows_v, sem).wait()
    pltpu.sync_copy(rows_v, out_hbm.at[pl.ds(base, CHUNK)])


def kernel(tokens, expert_ids, w1, w3, w2):
    eids = expert_ids.astype(jnp.int32)
    x_pad, dest, nblk = _route_scatter(eids, tokens)
    out_pad = _ffn(x_pad, nblk, w1, w3, w2)
    return _unpermute(dest, out_pad)


# speedup vs baseline: 1.0319x; 1.0021x over previous
import functools

import jax
import jax.numpy as jnp
from jax import lax
from jax.experimental import pallas as pl
from jax.experimental.pallas import tpu as pltpu
from jax.experimental.pallas import tpu_sc as plsc

E = 8
D = 1024
F = 4096
T = 2048
C = 384
EC = E * C
PAD_ROWS = (E + 1) * C

NC = 2
NS = 16
NW = NC * NS
CHUNK = T // NW
NG = CHUNK // 16

_MESH = dict(core_axis_name="c", subcore_axis_name="s",
             num_cores=NC, num_subcores=NS)


@functools.partial(
    pl.kernel,
    out_type=(
        jax.ShapeDtypeStruct((PAD_ROWS, D), jnp.float32),
        jax.ShapeDtypeStruct((T,), jnp.int32),
    ),
    mesh=plsc.VectorSubcoreMesh(**_MESH),
    compiler_params=pltpu.CompilerParams(needs_layout_passes=False),
    scratch_types=[
        pltpu.VMEM((T,), jnp.int32),
        pltpu.VMEM((CHUNK, D), jnp.float32),
        pltpu.VMEM((CHUNK,), jnp.int32),
        pltpu.SemaphoreType.DMA,
        pltpu.SemaphoreType.DMA,
    ],
)
def _route_scatter(eids_hbm, tokens_hbm, xpad_hbm, dest_hbm,
                   eids_v, rows_v, dest_v, sem_in, sem_out):
    wid = lax.axis_index("s") * NC + lax.axis_index("c")
    base = wid * CHUNK

    rows_cp = pltpu.make_async_copy(tokens_hbm.at[pl.ds(base, CHUNK)],
                                    rows_v, sem_in)
    rows_cp.start()
    pltpu.sync_copy(eids_hbm, eids_v)

    zero16 = jnp.zeros((16,), jnp.int32)

    def scan_body(g, carry):
        vec = eids_v[pl.ds(g * 16, 16)]
        return tuple(carry[e] + jnp.sum(jnp.where(vec == e, 1, 0))
                     for e in range(E))

    bases = lax.fori_loop(0, wid * NG, scan_body,
                          tuple(jnp.int32(0) for _ in range(E)))

    for g in range(NG):
        vec = eids_v[pl.ds(base + g * 16, 16)]
        pos = zero16
        new_bases = []
        for e in range(E):
            m = vec == e
            onehot = jnp.where(m, 1, 0).astype(jnp.int32)
            csum = plsc.cumsum(onehot)
            pos = jnp.where(m, csum - 1 + bases[e], pos)
            new_bases.append(bases[e] + jnp.sum(onehot))
        bases = tuple(new_bases)
        valid = pos < C
        dest = jnp.where(valid, vec * C + pos, EC + ((pos - C) % C))
        dest_v[pl.ds(g * 16, 16)] = dest

    pltpu.sync_copy(dest_v, dest_hbm.at[pl.ds(base, CHUNK)])
    rows_cp.wait()
    pltpu.async_copy(rows_v, xpad_hbm.at[dest_v], sem_out).wait()


FB = 2048
NF = F // FB
MB = 128
NM = C // MB


def _ffn_body(x_ref, w1_ref, w3_ref, w2_ref, out_ref):
    e = pl.program_id(0)
    f = pl.program_id(1)

    SB = 1024

    @pl.when(e < E)
    def _():
        x = x_ref[...].astype(jnp.bfloat16)
        contrib = jnp.zeros((C, D), jnp.float32)
        for s in range(FB // SB):
            w1b = w1_ref[0, :, pl.ds(s * SB, SB)].astype(jnp.bfloat16)
            w3b = w3_ref[0, :, pl.ds(s * SB, SB)].astype(jnp.bfloat16)
            w2b = w2_ref[0, pl.ds(s * SB, SB), :].astype(jnp.bfloat16)
            g = jnp.dot(x, w1b, preferred_element_type=jnp.float32)
            u = jnp.dot(x, w3b, preferred_element_type=jnp.float32)
            h = (g * jax.nn.sigmoid(g) * u).astype(jnp.bfloat16)
            contrib = contrib + jnp.dot(h, w2b,
                                        preferred_element_type=jnp.float32)

        @pl.when(f == 0)
        def _():
            out_ref[...] = contrib

        @pl.when(f > 0)
        def _():
            out_ref[...] = out_ref[...] + contrib

    @pl.when(jnp.logical_and(e == E, f == 0))
    def _():
        out_ref[...] = jnp.zeros_like(out_ref)


def _ffn(x_pad, w1, w3, w2):
    return pl.pallas_call(
        _ffn_body,
        grid=(E + 1, NF),
        in_specs=[
            pl.BlockSpec((C, D), lambda e, f: (jnp.minimum(e, E - 1), 0)),
            pl.BlockSpec((1, D, FB),
                         lambda e, f: (jnp.minimum(e, E - 1), 0,
                                       jnp.where(e == E, NF - 1, f))),
            pl.BlockSpec((1, D, FB),
                         lambda e, f: (jnp.minimum(e, E - 1), 0,
                                       jnp.where(e == E, NF - 1, f))),
            pl.BlockSpec((1, FB, D),
                         lambda e, f: (jnp.minimum(e, E - 1),
                                       jnp.where(e == E, NF - 1, f), 0)),
        ],
        out_specs=pl.BlockSpec((C, D), lambda e, f: (e, 0)),
        out_shape=jax.ShapeDtypeStruct((PAD_ROWS, D), jnp.float32),
    )(x_pad, w1, w3, w2)


@functools.partial(
    pl.kernel,
    out_type=jax.ShapeDtypeStruct((T, D), jnp.float32),
    mesh=plsc.VectorSubcoreMesh(**_MESH),
    compiler_params=pltpu.CompilerParams(needs_layout_passes=False),
    scratch_types=[
        pltpu.VMEM((CHUNK,), jnp.int32),
        pltpu.VMEM((CHUNK, D), jnp.float32),
        pltpu.SemaphoreType.DMA,
    ],
)
def _unpermute(dest_hbm, outpad_hbm, out_hbm, idx_v, rows_v, sem):
    wid = lax.axis_index("s") * NC + lax.axis_index("c")
    base = wid * CHUNK
    pltpu.sync_copy(dest_hbm.at[pl.ds(base, CHUNK)], idx_v)
    pltpu.async_copy(outpad_hbm.at[idx_v], rows_v, sem).wait()
    pltpu.sync_copy(rows_v, out_hbm.at[pl.ds(base, CHUNK)])


def kernel(tokens, expert_ids, w1, w3, w2):
    eids = expert_ids.astype(jnp.int32)
    x_pad, dest = _route_scatter(eids, tokens)
    out_pad = _ffn(x_pad, w1, w3, w2)
    return _unpermute(dest, out_pad)
